# baseline (speedup 1.0000x reference)
import jax
import jax.numpy as jnp
from jax.experimental import pallas as pl
from jax.experimental.pallas import tpu as pltpu

B, S, D, E, TOPK = 4, 8192, 768, 64, 2
BS = B * S
BM = 512


def _gating_kernel(xr_ref, xi_ref, wa_ref, wp_ref, b_ref, probs_ref, idx_ref):
    xr = xr_ref[...]
    xi = xi_ref[...]
    amp = jnp.sqrt(xr * xr + xi * xi)
    phase = jnp.arctan2(xi, xr)
    scores = (
        jnp.dot(amp, wa_ref[...], preferred_element_type=jnp.float32)
        + jnp.dot(phase, wp_ref[...], preferred_element_type=jnp.float32)
        + b_ref[...]
    )

    rlf = (63 - jax.lax.broadcasted_iota(jnp.int32, scores.shape, 1)).astype(
        jnp.float32)
    m1 = jnp.max(scores, axis=-1, keepdims=True)
    rl1 = jnp.max(jnp.where(scores == m1, rlf, -1.0), axis=-1, keepdims=True)
    masked = jnp.where(rlf == rl1, -jnp.inf, scores)
    m2 = jnp.max(masked, axis=-1, keepdims=True)
    rl2 = jnp.max(jnp.where(masked == m2, rlf, -1.0), axis=-1, keepdims=True)

    i1 = 63 - rl1.astype(jnp.int32)
    i2 = 63 - rl2.astype(jnp.int32)
    e = jnp.exp(m2 - m1)
    p0 = 1.0 / (1.0 + e)
    probs_ref[:, 0:1] = p0
    probs_ref[:, 1:2] = 1.0 - p0
    idx_ref[:, 0:1] = i1
    idx_ref[:, 1:2] = i2


@jax.jit
def kernel(x_real, x_imag, W, b):
    xr = x_real.reshape(BS, D)
    xi = x_imag.reshape(BS, D)
    wa = W[:D]
    wp = W[D:]
    b2 = b.reshape(1, E)

    grid = (BS // BM,)
    probs, idx = pl.pallas_call(
        _gating_kernel,
        grid=grid,
        in_specs=[
            pl.BlockSpec((BM, D), lambda i: (i, 0)),
            pl.BlockSpec((BM, D), lambda i: (i, 0)),
            pl.BlockSpec((D, E), lambda i: (0, 0)),
            pl.BlockSpec((D, E), lambda i: (0, 0)),
            pl.BlockSpec((1, E), lambda i: (0, 0)),
        ],
        out_specs=[
            pl.BlockSpec((BM, TOPK), lambda i: (i, 0)),
            pl.BlockSpec((BM, TOPK), lambda i: (i, 0)),
        ],
        out_shape=[
            jax.ShapeDtypeStruct((BS, TOPK), jnp.float32),
            jax.ShapeDtypeStruct((BS, TOPK), jnp.int32),
        ],
        compiler_params=pltpu.CompilerParams(
            dimension_semantics=("arbitrary",),
        ),
    )(xr, xi, wa, wp, b2)

    return probs.reshape(B, S, TOPK), idx.reshape(B, S, TOPK)


# speedup vs baseline: 1.0419x; 1.0419x over previous
import jax
import jax.numpy as jnp
from jax.experimental import pallas as pl
from jax.experimental.pallas import tpu as pltpu

B, S, D, E, TOPK = 4, 8192, 768, 64, 2
BS = B * S
BM = 1024


def _gating_kernel(xr_ref, xi_ref, wa_ref, wp_ref, b_ref, probs_ref, idx_ref):
    xr = xr_ref[...]
    xi = xi_ref[...]
    amp = jnp.sqrt(xr * xr + xi * xi)
    phase = jnp.arctan2(xi, xr)
    scores = (
        jnp.dot(amp, wa_ref[...], preferred_element_type=jnp.float32)
        + jnp.dot(phase, wp_ref[...], preferred_element_type=jnp.float32)
        + b_ref[...]
    )

    rlf = (63 - jax.lax.broadcasted_iota(jnp.int32, scores.shape, 1)).astype(
        jnp.float32)
    m1 = jnp.max(scores, axis=-1, keepdims=True)
    rl1 = jnp.max(jnp.where(scores == m1, rlf, -1.0), axis=-1, keepdims=True)
    masked = jnp.where(rlf == rl1, -jnp.inf, scores)
    m2 = jnp.max(masked, axis=-1, keepdims=True)
    rl2 = jnp.max(jnp.where(masked == m2, rlf, -1.0), axis=-1, keepdims=True)

    i1 = 63 - rl1.astype(jnp.int32)
    i2 = 63 - rl2.astype(jnp.int32)
    e = jnp.exp(m2 - m1)
    p0 = 1.0 / (1.0 + e)
    probs_ref[:, 0:1] = p0
    probs_ref[:, 1:2] = 1.0 - p0
    idx_ref[:, 0:1] = i1
    idx_ref[:, 1:2] = i2


@jax.jit
def kernel(x_real, x_imag, W, b):
    xr = x_real.reshape(BS, D)
    xi = x_imag.reshape(BS, D)
    wa = W[:D]
    wp = W[D:]
    b2 = b.reshape(1, E)

    grid = (BS // BM,)
    probs, idx = pl.pallas_call(
        _gating_kernel,
        grid=grid,
        in_specs=[
            pl.BlockSpec((BM, D), lambda i: (i, 0)),
            pl.BlockSpec((BM, D), lambda i: (i, 0)),
            pl.BlockSpec((D, E), lambda i: (0, 0)),
            pl.BlockSpec((D, E), lambda i: (0, 0)),
            pl.BlockSpec((1, E), lambda i: (0, 0)),
        ],
        out_specs=[
            pl.BlockSpec((BM, TOPK), lambda i: (i, 0)),
            pl.BlockSpec((BM, TOPK), lambda i: (i, 0)),
        ],
        out_shape=[
            jax.ShapeDtypeStruct((BS, TOPK), jnp.float32),
            jax.ShapeDtypeStruct((BS, TOPK), jnp.int32),
        ],
        compiler_params=pltpu.CompilerParams(
            dimension_semantics=("arbitrary",),
        ),
    )(xr, xi, wa, wp, b2)

    return probs.reshape(B, S, TOPK), idx.reshape(B, S, TOPK)


# speedup vs baseline: 1.0433x; 1.0014x over previous
import jax
import jax.numpy as jnp
from jax.experimental import pallas as pl
from jax.experimental.pallas import tpu as pltpu

B, S, D, E, TOPK = 4, 8192, 768, 64, 2
BS = B * S
BM = 2048


def _gating_kernel(xr_ref, xi_ref, wa_ref, wp_ref, b_ref, probs_ref, idx_ref):
    xr = xr_ref[...]
    xi = xi_ref[...]
    amp = jnp.sqrt(xr * xr + xi * xi)
    phase = jnp.arctan2(xi, xr)
    scores = (
        jnp.dot(amp, wa_ref[...], preferred_element_type=jnp.float32)
        + jnp.dot(phase, wp_ref[...], preferred_element_type=jnp.float32)
        + b_ref[...]
    )

    rlf = (63 - jax.lax.broadcasted_iota(jnp.int32, scores.shape, 1)).astype(
        jnp.float32)
    m1 = jnp.max(scores, axis=-1, keepdims=True)
    rl1 = jnp.max(jnp.where(scores == m1, rlf, -1.0), axis=-1, keepdims=True)
    masked = jnp.where(rlf == rl1, -jnp.inf, scores)
    m2 = jnp.max(masked, axis=-1, keepdims=True)
    rl2 = jnp.max(jnp.where(masked == m2, rlf, -1.0), axis=-1, keepdims=True)

    i1 = 63 - rl1.astype(jnp.int32)
    i2 = 63 - rl2.astype(jnp.int32)
    e = jnp.exp(m2 - m1)
    p0 = 1.0 / (1.0 + e)
    probs_ref[:, 0:1] = p0
    probs_ref[:, 1:2] = 1.0 - p0
    idx_ref[:, 0:1] = i1
    idx_ref[:, 1:2] = i2


@jax.jit
def kernel(x_real, x_imag, W, b):
    xr = x_real.reshape(BS, D)
    xi = x_imag.reshape(BS, D)
    wa = W[:D]
    wp = W[D:]
    b2 = b.reshape(1, E)

    grid = (BS // BM,)
    probs, idx = pl.pallas_call(
        _gating_kernel,
        grid=grid,
        in_specs=[
            pl.BlockSpec((BM, D), lambda i: (i, 0)),
            pl.BlockSpec((BM, D), lambda i: (i, 0)),
            pl.BlockSpec((D, E), lambda i: (0, 0)),
            pl.BlockSpec((D, E), lambda i: (0, 0)),
            pl.BlockSpec((1, E), lambda i: (0, 0)),
        ],
        out_specs=[
            pl.BlockSpec((BM, TOPK), lambda i: (i, 0)),
            pl.BlockSpec((BM, TOPK), lambda i: (i, 0)),
        ],
        out_shape=[
            jax.ShapeDtypeStruct((BS, TOPK), jnp.float32),
            jax.ShapeDtypeStruct((BS, TOPK), jnp.int32),
        ],
        compiler_params=pltpu.CompilerParams(
            dimension_semantics=("arbitrary",),
        ),
    )(xr, xi, wa, wp, b2)

    return probs.reshape(B, S, TOPK), idx.reshape(B, S, TOPK)
